# baseline (speedup 1.0000x reference)
import functools

import jax
import jax.numpy as jnp
from jax import lax
from jax.experimental import pallas as pl
from jax.experimental.pallas import tpu as pltpu
from jax.experimental.pallas import tpu_sc as plsc

N_NODES = 10000
N_EDGES = 320000
NP = 10240
STRIPE = 640
NC = 2
NS = 16
NW = NC * NS
CHUNK = 112
NCHUNK = 90
NE_PAD = NW * NCHUNK * CHUNK

_f32 = jnp.float32
_i32 = jnp.int32

_MESH = plsc.VectorSubcoreMesh(
    core_axis_name="c", subcore_axis_name="s", num_cores=NC, num_subcores=NS
)


def _make_deg_kernel():
    assert NCHUNK % 3 == 0

    def body(col3, ew3, dinit, out, col_b, ew_b, s0, s1, s2, acc):
        sems = (s0, s1, s2)
        c = lax.axis_index("c")
        s = lax.axis_index("s")
        w = c * NS + s
        pltpu.sync_copy(col3.at[w], col_b)
        pltpu.sync_copy(ew3.at[w], ew_b)
        stripe = pl.ds(pl.multiple_of(s * STRIPE, STRIPE), STRIPE)
        pltpu.sync_copy(dinit.at[c], acc.at[stripe])
        plsc.subcore_barrier()

        def scat(i, b):
            return pltpu.make_async_copy(
                ew_b.at[i], acc.at[col_b.at[i]], sems[b])

        def step(j, carry):
            for b in range(3):
                i = 3 * j + b

                @pl.when(j > 0)
                def _():
                    scat(i - 3, b).wait()

                scat(i, b).start(add=True)
            return carry

        lax.fori_loop(0, NCHUNK // 3, step, 0)
        for b in range(3):
            scat(NCHUNK - 3 + b, b).wait()
        plsc.subcore_barrier()
        pltpu.sync_copy(acc.at[stripe], out.at[c].at[stripe])

    return pl.kernel(
        body,
        out_type=jax.ShapeDtypeStruct((NC, NP), _f32),
        mesh=_MESH,
        scratch_types=[
            pltpu.VMEM((NCHUNK, CHUNK), _i32),
            pltpu.VMEM((NCHUNK, CHUNK), _f32),
            pltpu.SemaphoreType.DMA,
            pltpu.SemaphoreType.DMA,
            pltpu.SemaphoreType.DMA,
            pltpu.VMEM_SHARED((NP,), _f32),
        ],
        compiler_params=pltpu.CompilerParams(use_tc_tiling_on_sc=False),
    )


def _make_edge_kernel(d):
    assert NCHUNK % 3 == 0

    def body(hs, row3, col3, ew3, zsrc, out, row_b, col_b, ew_b,
             b0, b1, b2, g0, g1, g2, w0, w1, w2, acc):
        bufs = (b0, b1, b2)
        gsems = (g0, g1, g2)
        wsems = (w0, w1, w2)
        c = lax.axis_index("c")
        s = lax.axis_index("s")
        w = c * NS + s
        pltpu.sync_copy(row3.at[w], row_b)
        pltpu.sync_copy(col3.at[w], col_b)
        pltpu.sync_copy(ew3.at[w], ew_b)
        stripe = pl.ds(pl.multiple_of(s * STRIPE, STRIPE), STRIPE)

        @pl.when(c == 0)
        def _():
            pltpu.sync_copy(hs.at[stripe], acc.at[stripe])

        @pl.when(c != 0)
        def _():
            pltpu.sync_copy(zsrc, acc.at[stripe])

        plsc.subcore_barrier()

        def gather(i, b):
            return pltpu.make_async_copy(hs.at[row_b.at[i]], bufs[b], gsems[b])

        def scatter(i, b):
            return pltpu.make_async_copy(
                bufs[b], acc.at[col_b.at[i]], wsems[b])

        gather(0, 0).start()
        gather(1, 1).start()

        def step(j, carry):
            for b in range(3):
                i = 3 * j + b
                gather(i, b).wait()
                prv = (b - 1) % 3
                if b == 0:
                    @pl.when(j > 0)
                    def _():
                        scatter(i - 1, prv).wait()
                else:
                    scatter(i - 1, prv).wait()
                gather(jnp.minimum(i + 2, NCHUNK - 1), prv).start()
                buf = bufs[b]
                for g in range(CHUNK // 16):
                    ewv = ew_b[i, pl.ds(g * 16, 16)]
                    for el in range(16):
                        e = g * 16 + el
                        sp = jnp.full((16,), ewv[el])
                        for k in range(d // 16):
                            sl = pl.ds(k * 16, 16)
                            buf[e, sl] = buf[e, sl] * sp
                scatter(i, b).start(add=True)
            return carry

        lax.fori_loop(0, NCHUNK // 3, step, 0)
        gather(NCHUNK - 1, 0).wait()
        gather(NCHUNK - 1, 1).wait()
        scatter(NCHUNK - 1, 2).wait()
        plsc.subcore_barrier()
        pltpu.sync_copy(acc.at[stripe], out.at[c].at[stripe])

    return pl.kernel(
        body,
        out_type=jax.ShapeDtypeStruct((NC, NP, d), _f32),
        mesh=_MESH,
        scratch_types=[
            pltpu.VMEM((NCHUNK, CHUNK), _i32),
            pltpu.VMEM((NCHUNK, CHUNK), _i32),
            pltpu.VMEM((NCHUNK, CHUNK), _f32),
            pltpu.VMEM((CHUNK, d), _f32),
            pltpu.VMEM((CHUNK, d), _f32),
            pltpu.VMEM((CHUNK, d), _f32),
            pltpu.SemaphoreType.DMA,
            pltpu.SemaphoreType.DMA,
            pltpu.SemaphoreType.DMA,
            pltpu.SemaphoreType.DMA,
            pltpu.SemaphoreType.DMA,
            pltpu.SemaphoreType.DMA,
            pltpu.VMEM_SHARED((NP, d), _f32),
        ],
        compiler_params=pltpu.CompilerParams(use_tc_tiling_on_sc=False),
    )


BLK = 1280
GRID = NP // BLK


def _tc_first_body(degp_ref, x_ref, w_ref, hs_ref, dis_ref):
    deg = degp_ref[0] + degp_ref[1]
    dis = jnp.where(deg > 0, lax.rsqrt(deg), 0.0)
    dis_ref[...] = dis
    h = jnp.dot(x_ref[...], w_ref[...], preferred_element_type=_f32)
    hs_ref[...] = dis * h


def _tc_mid_body(parts_ref, dis_ref, b_ref, w_ref, hs_ref):
    acc = parts_ref[0] + parts_ref[1]
    dis = dis_ref[...]
    h = jnp.maximum(dis * acc + b_ref[...], 0.0)
    hs_ref[...] = dis * jnp.dot(h, w_ref[...], preferred_element_type=_f32)


def _tc_head_body(parts_ref, dis_ref, b_ref, w_ref, fcb_ref, out_ref):
    acc = parts_ref[0] + parts_ref[1]
    h = jnp.maximum(dis_ref[...] * acc + b_ref[...], 0.0)
    out_ref[...] = (
        jnp.dot(h, w_ref[...], preferred_element_type=_f32) + fcb_ref[...]
    )


def _full(shape):
    return pl.BlockSpec(shape, lambda i: tuple(0 for _ in shape))


def _tc_first(degp, x, w1):
    din, dout = w1.shape
    return pl.pallas_call(
        _tc_first_body,
        grid=(GRID,),
        in_specs=[
            pl.BlockSpec((NC, BLK, 1), lambda i: (0, i, 0)),
            pl.BlockSpec((BLK, din), lambda i: (i, 0)),
            _full((din, dout)),
        ],
        out_specs=[
            pl.BlockSpec((BLK, dout), lambda i: (i, 0)),
            pl.BlockSpec((BLK, 1), lambda i: (i, 0)),
        ],
        out_shape=[
            jax.ShapeDtypeStruct((NP, dout), _f32),
            jax.ShapeDtypeStruct((NP, 1), _f32),
        ],
    )(degp, x, w1)


def _tc_mid(parts, dis, b, w):
    din, dout = w.shape
    return pl.pallas_call(
        _tc_mid_body,
        grid=(GRID,),
        in_specs=[
            pl.BlockSpec((NC, BLK, din), lambda i: (0, i, 0)),
            pl.BlockSpec((BLK, 1), lambda i: (i, 0)),
            _full((1, din)),
            _full((din, dout)),
        ],
        out_specs=pl.BlockSpec((BLK, dout), lambda i: (i, 0)),
        out_shape=jax.ShapeDtypeStruct((NP, dout), _f32),
    )(parts, dis, b, w)


def _tc_head(parts, dis, b, w, fcb):
    din, dout = w.shape
    return pl.pallas_call(
        _tc_head_body,
        grid=(GRID,),
        in_specs=[
            pl.BlockSpec((NC, BLK, din), lambda i: (0, i, 0)),
            pl.BlockSpec((BLK, 1), lambda i: (i, 0)),
            _full((1, din)),
            _full((din, dout)),
            _full((1, dout)),
        ],
        out_specs=pl.BlockSpec((BLK, dout), lambda i: (i, 0)),
        out_shape=jax.ShapeDtypeStruct((NP, dout), _f32),
    )(parts, dis, b, w, fcb)


_deg_kernel = _make_deg_kernel()
_edge_kernel_64 = _make_edge_kernel(64)
_edge_kernel_32 = _make_edge_kernel(32)


@jax.jit
def kernel(x, edge_index, edge_weight, W1, b1, W2, b2, W3, b3, fcW, fcb):
    pad = NE_PAD - N_EDGES
    pad_ids = jnp.arange(pad, dtype=_i32) % N_NODES
    row3 = jnp.concatenate(
        [edge_index[0].astype(_i32), pad_ids]).reshape(NW, NCHUNK, CHUNK)
    col3 = jnp.concatenate(
        [edge_index[1].astype(_i32), pad_ids]).reshape(NW, NCHUNK, CHUNK)
    ew3 = jnp.pad(edge_weight, (0, pad)).reshape(NW, NCHUNK, CHUNK)
    x_pad = jnp.pad(x, ((0, NP - N_NODES), (0, 0)))
    dinit = jnp.stack(
        [jnp.ones((STRIPE,), _f32), jnp.zeros((STRIPE,), _f32)]
    )
    z64 = jnp.zeros((STRIPE, 64), _f32)
    z32 = jnp.zeros((STRIPE, 32), _f32)

    degp = _deg_kernel(col3, ew3, dinit)
    hs1, dis = _tc_first(degp.reshape(NC, NP, 1), x_pad, W1)
    parts1 = _edge_kernel_64(hs1, row3, col3, ew3, z64)
    hs2 = _tc_mid(parts1, dis, b1.reshape(1, 64), W2)
    parts2 = _edge_kernel_64(hs2, row3, col3, ew3, z64)
    hs3 = _tc_mid(parts2, dis, b2.reshape(1, 64), W3)
    parts3 = _edge_kernel_32(hs3, row3, col3, ew3, z32)
    out = _tc_head(parts3, dis, b3.reshape(1, 32), fcW, fcb.reshape(1, 1))
    return out[:N_NODES]


# speedup vs baseline: 1.0860x; 1.0860x over previous
import functools

import jax
import jax.numpy as jnp
from jax import lax
from jax.experimental import pallas as pl
from jax.experimental.pallas import tpu as pltpu
from jax.experimental.pallas import tpu_sc as plsc

N_NODES = 10000
N_EDGES = 320000
NP = 10240
STRIPE = 640
NC = 2
NS = 16
NW = NC * NS
CHUNK = 80
NCHUNK = 126
NE_PAD = NW * NCHUNK * CHUNK

_f32 = jnp.float32
_i32 = jnp.int32

_MESH = plsc.VectorSubcoreMesh(
    core_axis_name="c", subcore_axis_name="s", num_cores=NC, num_subcores=NS
)


def _make_deg_kernel():
    assert NCHUNK % 3 == 0

    def body(col3, ew3, dinit, out, col_b, ew_b, s0, s1, s2, acc):
        sems = (s0, s1, s2)
        c = lax.axis_index("c")
        s = lax.axis_index("s")
        w = c * NS + s
        pltpu.sync_copy(col3.at[w], col_b)
        pltpu.sync_copy(ew3.at[w], ew_b)
        stripe = pl.ds(pl.multiple_of(s * STRIPE, STRIPE), STRIPE)
        pltpu.sync_copy(dinit.at[c], acc.at[stripe])
        plsc.subcore_barrier()

        def scat(i, b):
            return pltpu.make_async_copy(
                ew_b.at[i], acc.at[col_b.at[i]], sems[b])

        def step(j, carry):
            for b in range(3):
                i = 3 * j + b

                @pl.when(j > 0)
                def _():
                    scat(i - 3, b).wait()

                scat(i, b).start(add=True)
            return carry

        lax.fori_loop(0, NCHUNK // 3, step, 0)
        for b in range(3):
            scat(NCHUNK - 3 + b, b).wait()
        plsc.subcore_barrier()
        pltpu.sync_copy(acc.at[stripe], out.at[c].at[stripe])

    return pl.kernel(
        body,
        out_type=jax.ShapeDtypeStruct((NC, NP), _f32),
        mesh=_MESH,
        scratch_types=[
            pltpu.VMEM((NCHUNK, CHUNK), _i32),
            pltpu.VMEM((NCHUNK, CHUNK), _f32),
            pltpu.SemaphoreType.DMA,
            pltpu.SemaphoreType.DMA,
            pltpu.SemaphoreType.DMA,
            pltpu.VMEM_SHARED((NP,), _f32),
        ],
        compiler_params=pltpu.CompilerParams(use_tc_tiling_on_sc=False),
    )


def _make_edge_kernel(d):
    assert NCHUNK % 3 == 0

    def body(hs, row3, col3, ew3, zsrc, out, row_b, col_b, ew_b,
             b0, b1, b2, g0, g1, g2, w0, w1, w2, acc):
        bufs = (b0, b1, b2)
        gsems = (g0, g1, g2)
        wsems = (w0, w1, w2)
        c = lax.axis_index("c")
        s = lax.axis_index("s")
        w = c * NS + s
        pltpu.sync_copy(row3.at[w], row_b)
        pltpu.sync_copy(col3.at[w], col_b)
        pltpu.sync_copy(ew3.at[w], ew_b)
        stripe = pl.ds(pl.multiple_of(s * STRIPE, STRIPE), STRIPE)

        @pl.when(c == 0)
        def _():
            pltpu.sync_copy(hs.at[stripe], acc.at[stripe])

        @pl.when(c != 0)
        def _():
            pltpu.sync_copy(zsrc, acc.at[stripe])

        plsc.subcore_barrier()

        def gather(i, b):
            return pltpu.make_async_copy(hs.at[row_b.at[i]], bufs[b], gsems[b])

        def scatter(i, b):
            return pltpu.make_async_copy(
                bufs[b], acc.at[col_b.at[i]], wsems[b])

        gather(0, 0).start()
        gather(1, 1).start()

        def step(j, carry):
            for b in range(3):
                i = 3 * j + b
                gather(i, b).wait()
                prv = (b - 1) % 3
                if b == 0:
                    @pl.when(j > 0)
                    def _():
                        scatter(i - 1, prv).wait()
                else:
                    scatter(i - 1, prv).wait()
                gather(jnp.minimum(i + 2, NCHUNK - 1), prv).start()
                buf = bufs[b]
                for g in range(CHUNK // 16):
                    ewv = ew_b[i, pl.ds(g * 16, 16)]
                    for el in range(16):
                        e = g * 16 + el
                        sp = jnp.full((16,), ewv[el])
                        for k in range(d // 16):
                            sl = pl.ds(k * 16, 16)
                            buf[e, sl] = buf[e, sl] * sp
                scatter(i, b).start(add=True)
            return carry

        lax.fori_loop(0, NCHUNK // 3, step, 0)
        gather(NCHUNK - 1, 0).wait()
        gather(NCHUNK - 1, 1).wait()
        scatter(NCHUNK - 1, 2).wait()
        plsc.subcore_barrier()
        pltpu.sync_copy(acc.at[stripe], out.at[c].at[stripe])

    return pl.kernel(
        body,
        out_type=jax.ShapeDtypeStruct((NC, NP, d), _f32),
        mesh=_MESH,
        scratch_types=[
            pltpu.VMEM((NCHUNK, CHUNK), _i32),
            pltpu.VMEM((NCHUNK, CHUNK), _i32),
            pltpu.VMEM((NCHUNK, CHUNK), _f32),
            pltpu.VMEM((CHUNK, d), _f32),
            pltpu.VMEM((CHUNK, d), _f32),
            pltpu.VMEM((CHUNK, d), _f32),
            pltpu.SemaphoreType.DMA,
            pltpu.SemaphoreType.DMA,
            pltpu.SemaphoreType.DMA,
            pltpu.SemaphoreType.DMA,
            pltpu.SemaphoreType.DMA,
            pltpu.SemaphoreType.DMA,
            pltpu.VMEM_SHARED((NP, d), _f32),
        ],
        compiler_params=pltpu.CompilerParams(use_tc_tiling_on_sc=False),
    )


BLK = 1280
GRID = NP // BLK


def _tc_first_body(degp_ref, x_ref, w_ref, hs_ref, dis_ref):
    deg = degp_ref[0] + degp_ref[1]
    dis = jnp.where(deg > 0, lax.rsqrt(deg), 0.0)
    dis_ref[...] = dis
    h = jnp.dot(x_ref[...], w_ref[...], preferred_element_type=_f32)
    hs_ref[...] = dis * h


def _tc_mid_body(parts_ref, dis_ref, b_ref, w_ref, hs_ref):
    acc = parts_ref[0] + parts_ref[1]
    dis = dis_ref[...]
    h = jnp.maximum(dis * acc + b_ref[...], 0.0)
    hs_ref[...] = dis * jnp.dot(h, w_ref[...], preferred_element_type=_f32)


def _tc_head_body(parts_ref, dis_ref, b_ref, w_ref, fcb_ref, out_ref):
    acc = parts_ref[0] + parts_ref[1]
    h = jnp.maximum(dis_ref[...] * acc + b_ref[...], 0.0)
    out_ref[...] = (
        jnp.dot(h, w_ref[...], preferred_element_type=_f32) + fcb_ref[...]
    )


def _full(shape):
    return pl.BlockSpec(shape, lambda i: tuple(0 for _ in shape))


def _tc_first(degp, x, w1):
    din, dout = w1.shape
    return pl.pallas_call(
        _tc_first_body,
        grid=(GRID,),
        in_specs=[
            pl.BlockSpec((NC, BLK, 1), lambda i: (0, i, 0)),
            pl.BlockSpec((BLK, din), lambda i: (i, 0)),
            _full((din, dout)),
        ],
        out_specs=[
            pl.BlockSpec((BLK, dout), lambda i: (i, 0)),
            pl.BlockSpec((BLK, 1), lambda i: (i, 0)),
        ],
        out_shape=[
            jax.ShapeDtypeStruct((NP, dout), _f32),
            jax.ShapeDtypeStruct((NP, 1), _f32),
        ],
    )(degp, x, w1)


def _tc_mid(parts, dis, b, w):
    din, dout = w.shape
    return pl.pallas_call(
        _tc_mid_body,
        grid=(GRID,),
        in_specs=[
            pl.BlockSpec((NC, BLK, din), lambda i: (0, i, 0)),
            pl.BlockSpec((BLK, 1), lambda i: (i, 0)),
            _full((1, din)),
            _full((din, dout)),
        ],
        out_specs=pl.BlockSpec((BLK, dout), lambda i: (i, 0)),
        out_shape=jax.ShapeDtypeStruct((NP, dout), _f32),
    )(parts, dis, b, w)


def _tc_head(parts, dis, b, w, fcb):
    din, dout = w.shape
    return pl.pallas_call(
        _tc_head_body,
        grid=(GRID,),
        in_specs=[
            pl.BlockSpec((NC, BLK, din), lambda i: (0, i, 0)),
            pl.BlockSpec((BLK, 1), lambda i: (i, 0)),
            _full((1, din)),
            _full((din, dout)),
            _full((1, dout)),
        ],
        out_specs=pl.BlockSpec((BLK, dout), lambda i: (i, 0)),
        out_shape=jax.ShapeDtypeStruct((NP, dout), _f32),
    )(parts, dis, b, w, fcb)


_deg_kernel = _make_deg_kernel()
_edge_kernel_64 = _make_edge_kernel(64)
_edge_kernel_32 = _make_edge_kernel(32)


@jax.jit
def kernel(x, edge_index, edge_weight, W1, b1, W2, b2, W3, b3, fcW, fcb):
    pad = NE_PAD - N_EDGES
    pad_ids = jnp.arange(pad, dtype=_i32) % N_NODES
    row3 = jnp.concatenate(
        [edge_index[0].astype(_i32), pad_ids]).reshape(NW, NCHUNK, CHUNK)
    col3 = jnp.concatenate(
        [edge_index[1].astype(_i32), pad_ids]).reshape(NW, NCHUNK, CHUNK)
    ew3 = jnp.pad(edge_weight, (0, pad)).reshape(NW, NCHUNK, CHUNK)
    x_pad = jnp.pad(x, ((0, NP - N_NODES), (0, 0)))
    dinit = jnp.stack(
        [jnp.ones((STRIPE,), _f32), jnp.zeros((STRIPE,), _f32)]
    )
    z64 = jnp.zeros((STRIPE, 64), _f32)
    z32 = jnp.zeros((STRIPE, 32), _f32)

    degp = _deg_kernel(col3, ew3, dinit)
    hs1, dis = _tc_first(degp.reshape(NC, NP, 1), x_pad, W1)
    parts1 = _edge_kernel_64(hs1, row3, col3, ew3, z64)
    hs2 = _tc_mid(parts1, dis, b1.reshape(1, 64), W2)
    parts2 = _edge_kernel_64(hs2, row3, col3, ew3, z64)
    hs3 = _tc_mid(parts2, dis, b2.reshape(1, 64), W3)
    parts3 = _edge_kernel_32(hs3, row3, col3, ew3, z32)
    out = _tc_head(parts3, dis, b3.reshape(1, 32), fcW, fcb.reshape(1, 1))
    return out[:N_NODES]
